# baseline (speedup 1.0000x reference)
import functools

import jax
import jax.numpy as jnp
from jax import lax
from jax.experimental import pallas as pl
from jax.experimental.pallas import tpu as pltpu
from jax.experimental.pallas import tpu_sc as plsc

NUM_CORES = 2
NUM_SUBCORES = 16
NUM_WORKERS = NUM_CORES * NUM_SUBCORES
LANES = 16

BATCH_N = 4096
SEQ = 200
DIM = 64
NCLS = 3
B_PER_W = BATCH_N // NUM_WORKERS
C1 = 128
C2 = SEQ - C1
DREG = DIM // LANES


def _sc_body(ids_hbm, table_hbm, wt_hbm, bv_hbm, out_hbm,
             idx_v, rows_v, wt_v, bv_v, out_v, sem):
    wid = lax.axis_index("s") * NUM_CORES + lax.axis_index("c")
    base = wid * B_PER_W

    pltpu.sync_copy(ids_hbm.at[pl.ds(base, B_PER_W)], idx_v)
    pltpu.sync_copy(wt_hbm, wt_v)
    pltpu.sync_copy(bv_hbm, bv_v)

    w = [[wt_v[c, pl.ds(k * LANES, LANES)] for k in range(DREG)]
         for c in range(NCLS)]
    bvec = [bv_v[c] for c in range(NCLS)]

    def per_example(i, carry):
        pltpu.async_copy(table_hbm.at[idx_v.at[i, pl.ds(0, C1)]],
                         rows_v.at[pl.ds(0, C1)], sem).wait()
        pltpu.async_copy(table_hbm.at[idx_v.at[i, pl.ds(C1, C2)]],
                         rows_v.at[pl.ds(C1, C2)], sem).wait()

        def accum(t, acc):
            return tuple(acc[k] + rows_v[t, pl.ds(k * LANES, LANES)]
                         for k in range(DREG))

        zero = jnp.zeros((LANES,), jnp.float32)
        acc = lax.fori_loop(0, SEQ, accum, (zero,) * DREG)
        lane = lax.iota(jnp.int32, LANES)
        res = zero
        for c in range(NCLS):
            t = bvec[c]
            for k in range(DREG):
                t = t + acc[k] * w[c][k]
            res = jnp.where(lane == c, jnp.full((LANES,), jnp.sum(t)), res)
        out_v[i] = res
        return carry

    lax.fori_loop(0, B_PER_W, per_example, 0)
    pltpu.sync_copy(out_v, out_hbm.at[pl.ds(base, B_PER_W)])


_sc_call = pl.kernel(
    _sc_body,
    out_type=jax.ShapeDtypeStruct((BATCH_N, LANES), jnp.float32),
    mesh=plsc.VectorSubcoreMesh(core_axis_name="c", subcore_axis_name="s"),
    compiler_params=pltpu.CompilerParams(
        needs_layout_passes=False, use_tc_tiling_on_sc=False),
    scratch_types=[
        pltpu.VMEM((B_PER_W, SEQ), jnp.int32),
        pltpu.VMEM((SEQ, DIM), jnp.float32),
        pltpu.VMEM((NCLS, DIM), jnp.float32),
        pltpu.VMEM((NCLS, LANES), jnp.float32),
        pltpu.VMEM((B_PER_W, LANES), jnp.float32),
        pltpu.SemaphoreType.DMA,
    ],
)


@jax.jit
def kernel(subword_ids, table, W, b):
    wt = (W.T / SEQ).astype(jnp.float32)
    bv = jnp.broadcast_to(b[:, None] / LANES, (NCLS, LANES))
    out = _sc_call(subword_ids, table, wt, jnp.asarray(bv, jnp.float32))
    return out[:, :NCLS]


# speedup vs baseline: 1.2745x; 1.2745x over previous
import functools

import jax
import jax.numpy as jnp
from jax import lax
from jax.experimental import pallas as pl
from jax.experimental.pallas import tpu as pltpu
from jax.experimental.pallas import tpu_sc as plsc

NUM_CORES = 2
NUM_SUBCORES = 16
NUM_WORKERS = NUM_CORES * NUM_SUBCORES
LANES = 16

BATCH_N = 4096
SEQ = 200
DIM = 64
NCLS = 3
B_PER_W = BATCH_N // NUM_WORKERS
C1 = 128
C2 = SEQ - C1
DREG = DIM // LANES


def _sc_body(ids_hbm, table_hbm, wt_hbm, bv_hbm, out_hbm,
             idx_v, rows0_v, rows1_v, wt_v, bv_v, out_v, sem0, sem1):
    wid = lax.axis_index("s") * NUM_CORES + lax.axis_index("c")
    base = wid * B_PER_W

    pltpu.sync_copy(ids_hbm.at[pl.ds(base, B_PER_W)], idx_v)
    pltpu.sync_copy(wt_hbm, wt_v)
    pltpu.sync_copy(bv_hbm, bv_v)

    w = [[wt_v[c, pl.ds(k * LANES, LANES)] for k in range(DREG)]
         for c in range(NCLS)]
    bvec = [bv_v[c] for c in range(NCLS)]

    def fire(i, buf, s):
        pltpu.async_copy(table_hbm.at[idx_v.at[i, pl.ds(0, C1)]],
                         buf.at[pl.ds(0, C1)], s)
        pltpu.async_copy(table_hbm.at[idx_v.at[i, pl.ds(C1, C2)]],
                         buf.at[pl.ds(C1, C2)], s)

    def drain(buf, s):
        pltpu.make_async_copy(table_hbm.at[pl.ds(0, SEQ)], buf, s).wait()

    zero = jnp.zeros((LANES,), jnp.float32)
    lane = lax.iota(jnp.int32, LANES)

    def process(i, buf, s):
        drain(buf, s)

        def accum(t, acc):
            return tuple(acc[k] + buf[t, pl.ds(k * LANES, LANES)]
                         for k in range(DREG))

        acc = lax.fori_loop(0, SEQ, accum, (zero,) * DREG, unroll=8)

        nxt = i + 2
        @pl.when(nxt < B_PER_W)
        def _():
            fire(nxt, buf, s)

        res = zero
        for c in range(NCLS):
            t = bvec[c]
            for k in range(DREG):
                t = t + acc[k] * w[c][k]
            res = jnp.where(lane == c, jnp.full((LANES,), jnp.sum(t)), res)
        out_v[i] = res

    fire(0, rows0_v, sem0)
    fire(1, rows1_v, sem1)

    def pair(j, carry):
        process(2 * j, rows0_v, sem0)
        process(2 * j + 1, rows1_v, sem1)
        return carry

    lax.fori_loop(0, B_PER_W // 2, pair, 0)
    pltpu.sync_copy(out_v, out_hbm.at[pl.ds(base, B_PER_W)])


_sc_call = pl.kernel(
    _sc_body,
    out_type=jax.ShapeDtypeStruct((BATCH_N, LANES), jnp.float32),
    mesh=plsc.VectorSubcoreMesh(core_axis_name="c", subcore_axis_name="s"),
    compiler_params=pltpu.CompilerParams(
        needs_layout_passes=False, use_tc_tiling_on_sc=False),
    scratch_types=[
        pltpu.VMEM((B_PER_W, SEQ), jnp.int32),
        pltpu.VMEM((SEQ, DIM), jnp.float32),
        pltpu.VMEM((SEQ, DIM), jnp.float32),
        pltpu.VMEM((NCLS, DIM), jnp.float32),
        pltpu.VMEM((NCLS, LANES), jnp.float32),
        pltpu.VMEM((B_PER_W, LANES), jnp.float32),
        pltpu.SemaphoreType.DMA,
        pltpu.SemaphoreType.DMA,
    ],
)


@jax.jit
def kernel(subword_ids, table, W, b):
    wt = (W.T / SEQ).astype(jnp.float32)
    bv = jnp.broadcast_to(b[:, None] / LANES, (NCLS, LANES))
    out = _sc_call(subword_ids, table, wt, jnp.asarray(bv, jnp.float32))
    return out[:, :NCLS]


# speedup vs baseline: 1.2786x; 1.0032x over previous
import functools

import jax
import jax.numpy as jnp
from jax import lax
from jax.experimental import pallas as pl
from jax.experimental.pallas import tpu as pltpu
from jax.experimental.pallas import tpu_sc as plsc

NUM_CORES = 2
NUM_SUBCORES = 16
NUM_WORKERS = NUM_CORES * NUM_SUBCORES
LANES = 16

BATCH_N = 4096
SEQ = 200
DIM = 64
NCLS = 3
B_PER_W = BATCH_N // NUM_WORKERS
IDS_PER_W = B_PER_W * SEQ
C1 = 128
C2 = SEQ - C1
DREG = DIM // LANES
OUT_W = LANES


def _sc_body(ids_hbm, table_hbm, wt_hbm, bv_hbm, out_hbm,
             idx_v, rows0_v, rows1_v, wt_v, bv_v, out_v, sem0, sem1):
    wid = lax.axis_index("s") * NUM_CORES + lax.axis_index("c")
    base = wid * B_PER_W

    pltpu.sync_copy(ids_hbm.at[pl.ds(base * SEQ, IDS_PER_W)], idx_v)
    pltpu.sync_copy(wt_hbm, wt_v)
    pltpu.sync_copy(bv_hbm, bv_v)

    w = [[wt_v[pl.ds(c * DIM + k * LANES, LANES)] for k in range(DREG)]
         for c in range(NCLS)]
    bvec = [bv_v[pl.ds(c * LANES, LANES)] for c in range(NCLS)]

    def fire(i, buf, s):
        pltpu.async_copy(table_hbm.at[idx_v.at[pl.ds(i * SEQ, C1)]],
                         buf.at[pl.ds(0, C1)], s)
        pltpu.async_copy(table_hbm.at[idx_v.at[pl.ds(i * SEQ + C1, C2)]],
                         buf.at[pl.ds(C1, C2)], s)

    def drain(buf, s):
        pltpu.make_async_copy(table_hbm.at[pl.ds(0, SEQ)], buf, s).wait()

    zero = jnp.zeros((LANES,), jnp.float32)
    lane = lax.iota(jnp.int32, LANES)

    def process(i, buf, s):
        drain(buf, s)

        def accum(t, acc):
            return tuple(acc[k] + buf[t, pl.ds(k * LANES, LANES)]
                         for k in range(DREG))

        acc = lax.fori_loop(0, SEQ, accum, (zero,) * DREG, unroll=8)

        nxt = i + 2
        @pl.when(nxt < B_PER_W)
        def _():
            fire(nxt, buf, s)

        res = zero
        for c in range(NCLS):
            t = bvec[c]
            for k in range(DREG):
                t = t + acc[k] * w[c][k]
            res = jnp.where(lane == c, jnp.full((LANES,), jnp.sum(t)), res)
        out_v[pl.ds(i * OUT_W, OUT_W)] = res

    fire(0, rows0_v, sem0)
    fire(1, rows1_v, sem1)

    def pair(j, carry):
        process(2 * j, rows0_v, sem0)
        process(2 * j + 1, rows1_v, sem1)
        return carry

    lax.fori_loop(0, B_PER_W // 2, pair, 0)
    pltpu.sync_copy(out_v, out_hbm.at[pl.ds(base * OUT_W, B_PER_W * OUT_W)])


_sc_call = pl.kernel(
    _sc_body,
    out_type=jax.ShapeDtypeStruct((BATCH_N * OUT_W,), jnp.float32),
    mesh=plsc.VectorSubcoreMesh(core_axis_name="c", subcore_axis_name="s"),
    compiler_params=pltpu.CompilerParams(
        needs_layout_passes=False, use_tc_tiling_on_sc=False),
    scratch_types=[
        pltpu.VMEM((IDS_PER_W,), jnp.int32),
        pltpu.VMEM((SEQ, DIM), jnp.float32),
        pltpu.VMEM((SEQ, DIM), jnp.float32),
        pltpu.VMEM((NCLS * DIM,), jnp.float32),
        pltpu.VMEM((NCLS * LANES,), jnp.float32),
        pltpu.VMEM((B_PER_W * OUT_W,), jnp.float32),
        pltpu.SemaphoreType.DMA,
        pltpu.SemaphoreType.DMA,
    ],
)


@jax.jit
def kernel(subword_ids, table, W, b):
    wt = (W.T / SEQ).astype(jnp.float32).reshape(-1)
    bv = jnp.broadcast_to(b[:, None] / LANES,
                          (NCLS, LANES)).astype(jnp.float32).reshape(-1)
    out = _sc_call(subword_ids.reshape(-1), table, wt, bv)
    return out.reshape(BATCH_N, OUT_W)[:, :NCLS]
